# baseline (speedup 1.0000x reference)
import functools

import jax
import jax.numpy as jnp
from jax import lax
from jax.experimental import pallas as pl
from jax.experimental.pallas import tpu as pltpu
from jax.experimental.pallas import tpu_sc as plsc

D = 512
TOTAL = 16384
SEG = 32
NSEG = TOTAL // SEG
LANES = 16
GSIZE = LANES * SEG

_mesh = plsc.VectorSubcoreMesh(core_axis_name="c", subcore_axis_name="s")
_NW = _mesh.num_cores * _mesh.num_subcores
_ROWS_PER_W = D // _NW
_NBUF = 4


@functools.partial(
    pl.kernel,
    out_type=jax.ShapeDtypeStruct((D, NSEG), jnp.float32),
    mesh=_mesh,
    scratch_types=[
        pltpu.VMEM((TOTAL,), jnp.float32),
        pltpu.VMEM((TOTAL,), jnp.float32),
        pltpu.VMEM((TOTAL,), jnp.float32),
        pltpu.VMEM((TOTAL,), jnp.float32),
        pltpu.VMEM((_ROWS_PER_W, NSEG), jnp.float32),
        pltpu.SemaphoreType.DMA,
        pltpu.SemaphoreType.DMA,
        pltpu.SemaphoreType.DMA,
        pltpu.SemaphoreType.DMA,
    ],
    compiler_params=pltpu.CompilerParams(needs_layout_passes=False),
)
def _prod_at(x_hbm, out_hbm, b0, b1, b2, b3, out_buf, s0, s1, s2, s3):
    wid = lax.axis_index("s") * _mesh.num_cores + lax.axis_index("c")
    row0 = wid * _ROWS_PER_W
    lane = lax.iota(jnp.int32, LANES)
    idx = [lane * SEG + ((lane + k) & (SEG - 1)) for k in range(SEG)]
    bufs = (b0, b1, b2, b3)
    sems = (s0, s1, s2, s3)

    copies = [pltpu.async_copy(x_hbm.at[row0 + r], bufs[r], sems[r])
              for r in range(_NBUF)]
    for r in range(_ROWS_PER_W):
        p = r % _NBUF
        copies[r].wait()

        @plsc.parallel_loop(0, NSEG // LANES, unroll=2)
        def group_body(g, p=p, r=r):
            off = g * GSIZE
            vals = [plsc.load_gather(bufs[p], [idx[k] + off])
                    for k in range(SEG)]
            while len(vals) > 1:
                vals = [vals[i] * vals[i + 1] for i in range(0, len(vals), 2)]
            plsc.store_scatter(out_buf, [lane * 0 + r, lane + g * LANES],
                               vals[0])
        if r + _NBUF < _ROWS_PER_W:
            copies.append(pltpu.async_copy(
                x_hbm.at[row0 + r + _NBUF], bufs[p], sems[p]))

    pltpu.sync_copy(out_buf, out_hbm.at[pl.ds(row0, _ROWS_PER_W), :])


def kernel(x):
    return _prod_at(x)


# speedup vs baseline: 1.1886x; 1.1886x over previous
import functools

import jax
import jax.numpy as jnp
from jax import lax
from jax.experimental import pallas as pl
from jax.experimental.pallas import tpu as pltpu
from jax.experimental.pallas import tpu_sc as plsc

D = 512
TOTAL = 16384
SEG = 32
NSEG = TOTAL // SEG
LANES = 16
GSIZE = LANES * SEG

_mesh = plsc.VectorSubcoreMesh(core_axis_name="c", subcore_axis_name="s")
_NW = _mesh.num_cores * _mesh.num_subcores
_ROWS_PER_W = D // _NW
_NBUF = 4


@functools.partial(
    pl.kernel,
    out_type=jax.ShapeDtypeStruct((D, NSEG), jnp.float32),
    mesh=_mesh,
    scratch_types=[
        pltpu.VMEM((TOTAL,), jnp.float32),
        pltpu.VMEM((TOTAL,), jnp.float32),
        pltpu.VMEM((TOTAL,), jnp.float32),
        pltpu.VMEM((TOTAL,), jnp.float32),
        pltpu.VMEM((_ROWS_PER_W, NSEG), jnp.float32),
        pltpu.SemaphoreType.DMA,
        pltpu.SemaphoreType.DMA,
        pltpu.SemaphoreType.DMA,
        pltpu.SemaphoreType.DMA,
    ],
    compiler_params=pltpu.CompilerParams(needs_layout_passes=False),
)
def _prod_at(x_hbm, out_hbm, b0, b1, b2, b3, out_buf, s0, s1, s2, s3):
    wid = lax.axis_index("s") * _mesh.num_cores + lax.axis_index("c")
    row0 = wid * _ROWS_PER_W
    lane = lax.iota(jnp.int32, LANES)
    idx = [lane * SEG + ((lane + k) & (SEG - 1)) for k in range(SEG)]
    bufs = (b0, b1, b2, b3)
    sems = (s0, s1, s2, s3)

    copies = [pltpu.async_copy(x_hbm.at[row0 + r], bufs[r], sems[r])
              for r in range(_NBUF)]
    for r in range(_ROWS_PER_W):
        p = r % _NBUF
        copies[r].wait()

        @plsc.parallel_loop(0, NSEG // LANES)
        def group_body(g, p=p, r=r):
            off = g * GSIZE
            vals = [plsc.load_gather(bufs[p], [idx[k] + off])
                    for k in range(SEG)]
            while len(vals) > 1:
                vals = [vals[i] * vals[i + 1] for i in range(0, len(vals), 2)]
            plsc.store_scatter(out_buf, [lane * 0 + r, lane + g * LANES],
                               vals[0])
        if r + _NBUF < _ROWS_PER_W:
            copies.append(pltpu.async_copy(
                x_hbm.at[row0 + r + _NBUF], bufs[p], sems[p]))

    pltpu.sync_copy(out_buf, out_hbm.at[pl.ds(row0, _ROWS_PER_W), :])


def kernel(x):
    return _prod_at(x)


# speedup vs baseline: 1.2357x; 1.0396x over previous
import functools

import jax
import jax.numpy as jnp
from jax import lax
from jax.experimental import pallas as pl
from jax.experimental.pallas import tpu as pltpu
from jax.experimental.pallas import tpu_sc as plsc

D = 512
TOTAL = 16384
SEG = 32
NSEG = TOTAL // SEG
LANES = 16
GSIZE = LANES * SEG

_mesh = plsc.VectorSubcoreMesh(core_axis_name="c", subcore_axis_name="s")
_NW = _mesh.num_cores * _mesh.num_subcores
_ROWS_PER_W = D // _NW
_NBUF = 4


@functools.partial(
    pl.kernel,
    out_type=jax.ShapeDtypeStruct((D, NSEG), jnp.float32),
    mesh=_mesh,
    scratch_types=[
        pltpu.VMEM((TOTAL,), jnp.float32),
        pltpu.VMEM((TOTAL,), jnp.float32),
        pltpu.VMEM((TOTAL,), jnp.float32),
        pltpu.VMEM((TOTAL,), jnp.float32),
        pltpu.VMEM((_ROWS_PER_W, NSEG), jnp.float32),
        pltpu.SemaphoreType.DMA,
        pltpu.SemaphoreType.DMA,
        pltpu.SemaphoreType.DMA,
        pltpu.SemaphoreType.DMA,
    ],
    compiler_params=pltpu.CompilerParams(needs_layout_passes=False),
)
def _prod_at(x_hbm, out_hbm, b0, b1, b2, b3, out_buf, s0, s1, s2, s3):
    wid = lax.axis_index("s") * _mesh.num_cores + lax.axis_index("c")
    row0 = wid * _ROWS_PER_W
    lane = lax.iota(jnp.int32, LANES)
    bufs = (b0, b1, b2, b3)
    sems = (s0, s1, s2, s3)

    swap_perm = {d: lane ^ d for d in (8, 4, 2, 1)}
    swap_mask = {d: (lane & d) == 0 for d in (8, 4, 2, 1)}
    rev = (((lane & 1) << 3) | ((lane & 2) << 1)
           | ((lane & 4) >> 1) | ((lane & 8) >> 3))

    _dnums = lax.GatherDimensionNumbers(
        offset_dims=(), collapsed_slice_dims=(0,), start_index_map=(0,))

    def _swap(v, d):
        return lax.gather(v, swap_perm[d][:, None], _dnums, (1,),
                          mode=lax.GatherScatterMode.PROMISE_IN_BOUNDS)

    copies = [pltpu.async_copy(x_hbm.at[row0 + r], bufs[r], sems[r])
              for r in range(_NBUF)]
    for r in range(_ROWS_PER_W):
        p = r % _NBUF
        copies[r].wait()

        @plsc.parallel_loop(0, NSEG // LANES)
        def group_body(g, p=p, r=r):
            off = g * GSIZE
            vals = [bufs[p][pl.ds(off + SEG * j, LANES)]
                    * bufs[p][pl.ds(off + SEG * j + LANES, LANES)]
                    for j in range(LANES)]
            for d in (8, 4, 2, 1):
                m = swap_mask[d]
                vals = [jnp.where(m,
                                  vals[2 * j] * _swap(vals[2 * j], d),
                                  vals[2 * j + 1] * _swap(vals[2 * j + 1], d))
                        for j in range(len(vals) // 2)]
            plsc.store_scatter(out_buf, [lane * 0 + r, rev + g * LANES],
                               vals[0])
        if r + _NBUF < _ROWS_PER_W:
            copies.append(pltpu.async_copy(
                x_hbm.at[row0 + r + _NBUF], bufs[p], sems[p]))

    pltpu.sync_copy(out_buf, out_hbm.at[pl.ds(row0, _ROWS_PER_W), :])


def kernel(x):
    return _prod_at(x)
